# baseline (speedup 1.0000x reference)
import functools

import jax
import jax.numpy as jnp
from jax import lax
from jax.experimental import pallas as pl
from jax.experimental.pallas import tpu as pltpu
from jax.experimental.pallas import tpu_sc as plsc

NC = 2
NS = 16
L = 16
NW = NC * NS

V = 1000000
B = 16384
B2 = 2 * B
D = 64
BPW = B // NW
NBLK = (V + 127) // 128
SLAB = 244
CHW = 512
NCH = 61
TAILC = 7812 * 128
CAP = 24576
NIT = B2 // L


def _extract_body(tgt_hbm, ctx_hbm, tablet_hbm, tail_hbm, stage_hbm,
                  qv, mv, mj, chunk_v, tail_v, ccol_v, cj_v, rowg_v, sem_r):
    wid = lax.axis_index("s") * NC + lax.axis_index("c")
    sb = wid * SLAB
    se = jnp.where(wid == NW - 1, NBLK, sb + SLAB)

    pltpu.sync_copy(tgt_hbm, qv.at[pl.ds(0, B)])
    pltpu.sync_copy(ctx_hbm, qv.at[pl.ds(B, B)])

    @pl.when(wid == NW - 1)
    def _():
        pltpu.sync_copy(tail_hbm, tail_v)

    iota = lax.iota(jnp.int32, L)

    def extract_from(cv, lo, hi, cnt):
        def group_body(e, carry):
            base = e * L
            ev = mv[pl.ds(base, L)]
            ej = mj[pl.ds(base, L)]
            inlist = iota < (cnt - base)
            mc = inlist & (ev >= lo) & (ev < hi)
            nmc = plsc.all_reduce_population_count(mc)[0]

            @pl.when(nmc > 16)
            def _():
                plsc.store_compressed(ccol_v.at[pl.ds(0, L)], ev - lo, mask=mc)
                plsc.store_compressed(cj_v.at[pl.ds(0, L)], ej, mask=mc)
                ccol = ccol_v[pl.ds(0, L)]
                cj16 = cj_v[pl.ds(0, L)]
                vmask = iota < nmc
                rb = (e % 2) * (L * D)
                for f in range(D):
                    f16 = jnp.full((L,), f, jnp.int32)
                    g = plsc.load_gather(cv, [f16, ccol], mask=vmask)
                    plsc.store_scatter(
                        rowg_v, [rb + iota * D + f], g, mask=vmask)

                def fire(k, c2):
                    jk = cj16[jnp.broadcast_to(k, (L,))][0]
                    pltpu.async_copy(
                        rowg_v.at[pl.ds(rb + k * D, D)],
                        stage_hbm.at[pl.ds(jk * D, D)], sem_r)
                    return c2

                lax.fori_loop(0, nmc, fire, 0)

                def drain(k, c2):
                    pltpu.make_async_copy(
                        stage_hbm.at[pl.ds(0, D)],
                        rowg_v.at[pl.ds(rb, D)], sem_r).wait()
                    return c2

                lax.fori_loop(0, nmc, drain, 0)

            return carry

        lax.fori_loop(0, (cnt + L - 1) // L, group_body, 0)

    def round_body(carry):
        it0, _ = carry

        def scan_cond(c):
            it, cnt = c
            return (it < NIT) & (cnt <= CAP - L)

        def scan_step(c):
            it, cnt = c
            v16 = qv[pl.ds(it * L, L)]
            blk = lax.shift_right_logical(v16, 7)
            m = (blk >= sb) & (blk < se)
            plsc.store_compressed(mv.at[pl.ds(cnt, L)], v16, mask=m)
            plsc.store_compressed(
                mj.at[pl.ds(cnt, L)], it * L + iota, mask=m)
            return it + 1, cnt + plsc.all_reduce_population_count(m)[0]

        it1, cnt = lax.while_loop(scan_cond, scan_step, (it0, 0))

        def chunk_body(cc, c2):
            cst = pl.multiple_of((sb * 128 + cc * CHW) // 128, 1) * 128
            cst = pl.multiple_of(cst, 128)
            pltpu.sync_copy(tablet_hbm.at[:, pl.ds(cst, CHW)], chunk_v)
            extract_from(chunk_v, cst, cst + CHW, cnt)
            return c2

        lax.fori_loop(0, NCH, chunk_body, 0)

        @pl.when(wid == NW - 1)
        def _():
            cst = pl.multiple_of(7808 * 128, 128)
            pltpu.sync_copy(tablet_hbm.at[:, pl.ds(cst, CHW)], chunk_v)
            extract_from(chunk_v, cst, cst + CHW, cnt)
            extract_from(tail_v, TAILC, V, cnt)

        return it1, cnt

    lax.while_loop(lambda c: c[0] < NIT, round_body, (0, 0))


_extract = functools.partial(
    pl.kernel,
    out_type=jax.ShapeDtypeStruct((B2 * D,), jnp.float32),
    mesh=plsc.VectorSubcoreMesh(
        core_axis_name="c", subcore_axis_name="s",
        num_cores=NC, num_subcores=NS),
    scratch_types=[
        pltpu.VMEM((B2,), jnp.int32),
        pltpu.VMEM((CAP,), jnp.int32),
        pltpu.VMEM((CAP,), jnp.int32),
        pltpu.VMEM((D, CHW), jnp.float32),
        pltpu.VMEM((D, D), jnp.float32),
        pltpu.VMEM((L,), jnp.int32),
        pltpu.VMEM((L,), jnp.int32),
        pltpu.VMEM((2 * L * D,), jnp.float32),
        pltpu.SemaphoreType.DMA,
    ],
    compiler_params=pltpu.CompilerParams(needs_layout_passes=False),
)(_extract_body)


def _dot_body(stage_hbm, out_hbm, trows_v, crows_v, out_v):
    wid = lax.axis_index("s") * NC + lax.axis_index("c")
    base = wid * BPW

    pltpu.sync_copy(stage_hbm.at[pl.ds(base * D, BPW * D)], trows_v)
    pltpu.sync_copy(stage_hbm.at[pl.ds((B + base) * D, BPW * D)], crows_v)

    iota = lax.iota(jnp.int32, L)

    def blk_body(blk, carry):
        v = jnp.zeros((L,), jnp.float32)
        for j in range(L):
            r = blk * L + j
            s = jnp.zeros((L,), jnp.float32)
            for d in range(0, D, L):
                tv = trows_v[pl.ds(r * D + d, L)]
                cv = crows_v[pl.ds(r * D + d, L)]
                s = s + tv * cv
            v = jnp.where(iota == j, jnp.sum(s), v)
        out_v[pl.ds(blk * L, L)] = 1.0 / (1.0 + jnp.exp(-v))
        return carry

    lax.fori_loop(0, BPW // L, blk_body, 0)
    pltpu.sync_copy(out_v, out_hbm.at[pl.ds(base, BPW)])


_dot = functools.partial(
    pl.kernel,
    out_type=jax.ShapeDtypeStruct((B,), jnp.float32),
    mesh=plsc.VectorSubcoreMesh(
        core_axis_name="c", subcore_axis_name="s",
        num_cores=NC, num_subcores=NS),
    scratch_types=[
        pltpu.VMEM((BPW * D,), jnp.float32),
        pltpu.VMEM((BPW * D,), jnp.float32),
        pltpu.VMEM((BPW,), jnp.float32),
    ],
    compiler_params=pltpu.CompilerParams(needs_layout_passes=False),
)(_dot_body)


@jax.jit
def kernel(target_i, context_j, label, shared_embedding):
    table_t = shared_embedding.T
    tail = lax.slice(table_t, (0, TAILC), (D, V))
    stage = _extract(target_i, context_j, table_t, tail)
    out = _dot(stage)
    return (out, label.astype(jnp.float32))


# speedup vs baseline: 1.0841x; 1.0841x over previous
import functools

import jax
import jax.numpy as jnp
from jax import lax
from jax.experimental import pallas as pl
from jax.experimental.pallas import tpu as pltpu
from jax.experimental.pallas import tpu_sc as plsc

NC = 2
NS = 16
L = 16
NW = NC * NS

V = 1000000
B = 16384
B2 = 2 * B
D = 64
BPW = B // NW
NBLK = (V + 127) // 128
SLAB = 244
CHW = 512
NCH = 61
TAILC = 7812 * 128
CAP = 12288
NIT = B2 // L
NBIN = 64


def _extract_body(tgt_hbm, ctx_hbm, tablet_hbm, tail_hbm, stage_hbm,
                  qv, mv, mj, sv, sj, chunk_v, tail_v, ccol_v, cj_v,
                  rowg_v, hist_s, segs_s, cur_s, sem_r):
    wid = lax.axis_index("s") * NC + lax.axis_index("c")
    sb = wid * SLAB
    sbc = sb * 128
    se = jnp.where(wid == NW - 1, NBLK, sb + SLAB)

    iota = lax.iota(jnp.int32, L)
    lane0 = iota == 0

    def extract_seg(cv, lo, s0, s1, ring):
        def group_body(e, ring):
            g0, p0, p1 = ring
            base = s0 + e * L
            ev = sv[pl.ds(base, L)]
            ej = sj[pl.ds(base, L)]
            nmc = jnp.minimum(s1 - base, L)
            vmask = iota < nmc
            h = g0 % 2
            rb = h * (L * D)
            pend = jnp.where(h == 0, p0, p1)

            def drain(k, c2):
                pltpu.make_async_copy(
                    stage_hbm.at[pl.ds(0, D)],
                    rowg_v.at[pl.ds(0, D)], sem_r).wait()
                return c2

            lax.fori_loop(0, pend, drain, 0)

            cols = ev - lo
            for f in range(D):
                f16 = jnp.full((L,), f, jnp.int32)
                g = plsc.load_gather(cv, [f16, cols], mask=vmask)
                plsc.store_scatter(rowg_v, [rb + iota * D + f], g, mask=vmask)

            def fire(k, c2):
                jk = ej[jnp.broadcast_to(k, (L,))][0]
                pltpu.async_copy(
                    rowg_v.at[pl.ds(rb + k * D, D)],
                    stage_hbm.at[pl.ds(jk * D, D)], sem_r)
                return c2

            lax.fori_loop(0, nmc, fire, 0)
            p0n = jnp.where(h == 0, nmc, p0)
            p1n = jnp.where(h == 1, nmc, p1)
            return g0 + 1, p0n, p1n

        ngrp = jnp.maximum(s1 - s0 + L - 1, 0) // L
        return lax.fori_loop(0, ngrp, group_body, ring)

    def round_body(carry):
        it0, _ = carry
        pltpu.sync_copy(tgt_hbm, qv.at[pl.ds(0, B)])
        pltpu.sync_copy(ctx_hbm, qv.at[pl.ds(B, B)])

        def scan_cond(c):
            it, cnt = c
            return (it < NIT) & (cnt <= CAP - L)

        def scan_step(c):
            it, cnt = c
            v16 = qv[pl.ds(it * L, L)]
            blk = lax.shift_right_logical(v16, 7)
            m = (blk >= sb) & (blk < se)
            plsc.store_compressed(mv.at[pl.ds(cnt, L)], v16, mask=m)
            plsc.store_compressed(
                mj.at[pl.ds(cnt, L)], it * L + iota, mask=m)
            return it + 1, cnt + plsc.all_reduce_population_count(m)[0]

        it1, cnt = lax.while_loop(scan_cond, scan_step, (it0, 0))

        def zero_bin(k, c2):
            hist_s[k] = 0
            return c2

        lax.fori_loop(0, NBIN, zero_bin, 0)

        def hist_group(e, c2):
            base = e * L
            ev = mv[pl.ds(base, L)]
            for l in range(L):
                @pl.when(base + l < cnt)
                def _():
                    b = lax.shift_right_logical(ev[l] - sbc, 9)
                    hist_s[b] = hist_s[b] + 1
            return c2

        lax.fori_loop(0, (cnt + L - 1) // L, hist_group, 0)

        def prefix(k, acc):
            segs_s[k] = acc
            cur_s[k] = acc
            return acc + hist_s[k]

        total = lax.fori_loop(0, NBIN, prefix, 0)
        segs_s[NBIN] = total

        def scat_group(e, c2):
            base = e * L
            ev = mv[pl.ds(base, L)]
            ej = mj[pl.ds(base, L)]
            for l in range(L):
                @pl.when(base + l < cnt)
                def _():
                    b = lax.shift_right_logical(ev[l] - sbc, 9)
                    slot = cur_s[b]
                    cur_s[b] = slot + 1
                    plsc.store_scatter(
                        sv, [jnp.broadcast_to(slot, (L,))],
                        jnp.broadcast_to(ev[l], (L,)), mask=lane0)
                    plsc.store_scatter(
                        sj, [jnp.broadcast_to(slot, (L,))],
                        jnp.broadcast_to(ej[l], (L,)), mask=lane0)
            return c2

        lax.fori_loop(0, (cnt + L - 1) // L, scat_group, 0)

        def chunk_body(cc, ring):
            cst = pl.multiple_of(sbc + cc * CHW, 128)
            pltpu.sync_copy(tablet_hbm.at[:, pl.ds(cst, CHW)], chunk_v)
            return extract_seg(chunk_v, cst, segs_s[cc], segs_s[cc + 1], ring)

        ring = lax.fori_loop(0, NCH, chunk_body, (0, 0, 0))

        g0, p0, p1 = ring

        @pl.when(wid == NW - 1)
        def _():
            pltpu.sync_copy(tail_hbm, tail_v)
            cst = pl.multiple_of(7808 * 128, 128)
            pltpu.sync_copy(tablet_hbm.at[:, pl.ds(cst, CHW)], chunk_v)
            r2 = extract_seg(chunk_v, cst, segs_s[NCH], segs_s[NCH + 1],
                             (g0, p0, p1))
            r3 = extract_seg(tail_v, TAILC, segs_s[NCH + 1], segs_s[NCH + 2],
                             r2)
            ga, pa, pb = r3

            def drain(k, c2):
                pltpu.make_async_copy(
                    stage_hbm.at[pl.ds(0, D)],
                    rowg_v.at[pl.ds(0, D)], sem_r).wait()
                return c2

            lax.fori_loop(0, pa + pb, drain, 0)

        @pl.when(wid != NW - 1)
        def _():
            def drain(k, c2):
                pltpu.make_async_copy(
                    stage_hbm.at[pl.ds(0, D)],
                    rowg_v.at[pl.ds(0, D)], sem_r).wait()
                return c2

            lax.fori_loop(0, p0 + p1, drain, 0)

        return it1, cnt

    lax.while_loop(lambda c: c[0] < NIT, round_body, (0, 0))


_extract = functools.partial(
    pl.kernel,
    out_type=jax.ShapeDtypeStruct((B2 * D,), jnp.float32),
    mesh=plsc.VectorSubcoreMesh(
        core_axis_name="c", subcore_axis_name="s",
        num_cores=NC, num_subcores=NS),
    scratch_types=[
        pltpu.VMEM((B2,), jnp.int32),
        pltpu.VMEM((CAP,), jnp.int32),
        pltpu.VMEM((CAP,), jnp.int32),
        pltpu.VMEM((CAP,), jnp.int32),
        pltpu.VMEM((CAP,), jnp.int32),
        pltpu.VMEM((D, CHW), jnp.float32),
        pltpu.VMEM((D, D), jnp.float32),
        pltpu.VMEM((L,), jnp.int32),
        pltpu.VMEM((L,), jnp.int32),
        pltpu.VMEM((2 * L * D,), jnp.float32),
        pltpu.SMEM((NBIN,), jnp.int32),
        pltpu.SMEM((NBIN + 1,), jnp.int32),
        pltpu.SMEM((NBIN,), jnp.int32),
        pltpu.SemaphoreType.DMA,
    ],
    compiler_params=pltpu.CompilerParams(needs_layout_passes=False),
)(_extract_body)


def _dot_body(stage_hbm, out_hbm, trows_v, crows_v, out_v):
    wid = lax.axis_index("s") * NC + lax.axis_index("c")
    base = wid * BPW

    pltpu.sync_copy(stage_hbm.at[pl.ds(base * D, BPW * D)], trows_v)
    pltpu.sync_copy(stage_hbm.at[pl.ds((B + base) * D, BPW * D)], crows_v)

    iota = lax.iota(jnp.int32, L)

    def blk_body(blk, carry):
        v = jnp.zeros((L,), jnp.float32)
        for j in range(L):
            r = blk * L + j
            s = jnp.zeros((L,), jnp.float32)
            for d in range(0, D, L):
                tv = trows_v[pl.ds(r * D + d, L)]
                cv = crows_v[pl.ds(r * D + d, L)]
                s = s + tv * cv
            v = jnp.where(iota == j, jnp.sum(s), v)
        out_v[pl.ds(blk * L, L)] = 1.0 / (1.0 + jnp.exp(-v))
        return carry

    lax.fori_loop(0, BPW // L, blk_body, 0)
    pltpu.sync_copy(out_v, out_hbm.at[pl.ds(base, BPW)])


_dot = functools.partial(
    pl.kernel,
    out_type=jax.ShapeDtypeStruct((B,), jnp.float32),
    mesh=plsc.VectorSubcoreMesh(
        core_axis_name="c", subcore_axis_name="s",
        num_cores=NC, num_subcores=NS),
    scratch_types=[
        pltpu.VMEM((BPW * D,), jnp.float32),
        pltpu.VMEM((BPW * D,), jnp.float32),
        pltpu.VMEM((BPW,), jnp.float32),
    ],
    compiler_params=pltpu.CompilerParams(needs_layout_passes=False),
)(_dot_body)


@jax.jit
def kernel(target_i, context_j, label, shared_embedding):
    table_t = shared_embedding.T
    tail = lax.slice(table_t, (0, TAILC), (D, V))
    stage = _extract(target_i, context_j, table_t, tail)
    out = _dot(stage)
    return (out, label.astype(jnp.float32))


# speedup vs baseline: 1.3451x; 1.2408x over previous
import functools

import jax
import jax.numpy as jnp
from jax import lax
from jax.experimental import pallas as pl
from jax.experimental.pallas import tpu as pltpu
from jax.experimental.pallas import tpu_sc as plsc

NC = 2
NS = 16
L = 16
NW = NC * NS

V = 1000000
B = 16384
B2 = 2 * B
D = 64
BPW = B // NW
NBLK = (V + 127) // 128
SLAB = 244
CHW = 512
NCH = 61
TAILC = 7812 * 128
CAP = 4096
NIT = B2 // L
NBIN = 64


def _extract_body(tgt_hbm, ctx_hbm, tablet_hbm, tail_hbm, stage_hbm,
                  qv, mv, mj, sv, sj, chunk_v, tail_v,
                  rowg_v, hist_s, segs_s, cur_s, sem_r, sem_c0, sem_c1):
    wid = lax.axis_index("s") * NC + lax.axis_index("c")
    sb = wid * SLAB
    sbc = sb * 128
    se = jnp.where(wid == NW - 1, NBLK, sb + SLAB)

    iota = lax.iota(jnp.int32, L)
    lane0 = iota == 0

    def extract_seg(cv, lo, cbase, s0, s1, ring):
        def group_body(e, ring):
            g0, p0, p1 = ring
            base = s0 + e * L
            ev = sv[pl.ds(base, L)]
            ej = sj[pl.ds(base, L)]
            nmc = jnp.minimum(s1 - base, L)
            vmask = iota < nmc
            h = g0 % 2
            rb = h * (L * D)
            pend = jnp.where(h == 0, p0, p1)

            def drain(k, c2):
                pltpu.make_async_copy(
                    stage_hbm.at[pl.ds(0, D)],
                    rowg_v.at[pl.ds(0, D)], sem_r).wait()
                return c2

            lax.fori_loop(0, pend, drain, 0)

            cols = ev - lo + cbase
            for f in range(D):
                f16 = jnp.full((L,), f, jnp.int32)
                g = plsc.load_gather(cv, [f16, cols], mask=vmask)
                plsc.store_scatter(rowg_v, [rb + iota * D + f], g, mask=vmask)

            def fire(k, c2):
                jk = ej[jnp.broadcast_to(k, (L,))][0]
                pltpu.async_copy(
                    rowg_v.at[pl.ds(rb + k * D, D)],
                    stage_hbm.at[pl.ds(jk * D, D)], sem_r)
                return c2

            lax.fori_loop(0, nmc, fire, 0)
            p0n = jnp.where(h == 0, nmc, p0)
            p1n = jnp.where(h == 1, nmc, p1)
            return g0 + 1, p0n, p1n

        ngrp = jnp.maximum(s1 - s0 + L - 1, 0) // L
        return lax.fori_loop(0, ngrp, group_body, ring)

    def round_body(carry):
        it0, _ = carry
        pltpu.sync_copy(tgt_hbm, qv.at[pl.ds(0, B)])
        pltpu.sync_copy(ctx_hbm, qv.at[pl.ds(B, B)])

        def scan_cond(c):
            it, cnt = c
            return (it < NIT) & (cnt <= CAP - L)

        def scan_step(c):
            it, cnt = c
            v16 = qv[pl.ds(it * L, L)]
            blk = lax.shift_right_logical(v16, 7)
            m = (blk >= sb) & (blk < se)
            plsc.store_compressed(mv.at[pl.ds(cnt, L)], v16, mask=m)
            plsc.store_compressed(
                mj.at[pl.ds(cnt, L)], it * L + iota, mask=m)
            return it + 1, cnt + plsc.all_reduce_population_count(m)[0]

        it1, cnt = lax.while_loop(scan_cond, scan_step, (it0, 0))

        def zero_bin(k, c2):
            hist_s[k] = 0
            return c2

        lax.fori_loop(0, NBIN, zero_bin, 0)

        def hist_group(e, c2):
            base = e * L
            ev = mv[pl.ds(base, L)]
            for l in range(L):
                @pl.when(base + l < cnt)
                def _():
                    b = lax.shift_right_logical(ev[l] - sbc, 9)
                    hist_s[b] = hist_s[b] + 1
            return c2

        lax.fori_loop(0, (cnt + L - 1) // L, hist_group, 0)

        def prefix(k, acc):
            segs_s[k] = acc
            cur_s[k] = acc
            return acc + hist_s[k]

        total = lax.fori_loop(0, NBIN, prefix, 0)
        segs_s[NBIN] = total

        def scat_group(e, c2):
            base = e * L
            ev = mv[pl.ds(base, L)]
            ej = mj[pl.ds(base, L)]
            for l in range(L):
                @pl.when(base + l < cnt)
                def _():
                    b = lax.shift_right_logical(ev[l] - sbc, 9)
                    slot = cur_s[b]
                    cur_s[b] = slot + 1
                    plsc.store_scatter(
                        sv, [jnp.broadcast_to(slot, (L,))],
                        jnp.broadcast_to(ev[l], (L,)), mask=lane0)
                    plsc.store_scatter(
                        sj, [jnp.broadcast_to(slot, (L,))],
                        jnp.broadcast_to(ej[l], (L,)), mask=lane0)
            return c2

        lax.fori_loop(0, (cnt + L - 1) // L, scat_group, 0)

        def prefetch0(cc):
            cst = pl.multiple_of(sbc + cc * CHW, 128)
            pltpu.async_copy(tablet_hbm.at[:, pl.ds(cst, CHW)],
                             chunk_v.at[:, pl.ds(0, CHW)], sem_c0)

        def prefetch1(cc):
            cst = pl.multiple_of(sbc + cc * CHW, 128)
            pltpu.async_copy(tablet_hbm.at[:, pl.ds(cst, CHW)],
                             chunk_v.at[:, pl.ds(CHW, CHW)], sem_c1)

        def wait0():
            pltpu.make_async_copy(tablet_hbm.at[:, pl.ds(0, CHW)],
                                  chunk_v.at[:, pl.ds(0, CHW)], sem_c0).wait()

        def wait1():
            pltpu.make_async_copy(tablet_hbm.at[:, pl.ds(0, CHW)],
                                  chunk_v.at[:, pl.ds(CHW, CHW)], sem_c1).wait()

        prefetch0(0)

        def chunk_body(cc, ring):
            h = cc % 2
            cst = sbc + cc * CHW

            @pl.when(h == 0)
            def _():
                wait0()

            @pl.when(h == 1)
            def _():
                wait1()

            @pl.when((cc + 1 < NCH) & (h == 0))
            def _():
                prefetch1(cc + 1)

            @pl.when((cc + 1 < NCH) & (h == 1))
            def _():
                prefetch0(cc + 1)

            return extract_seg(chunk_v, cst, h * CHW,
                               segs_s[cc], segs_s[cc + 1], ring)

        ring = lax.fori_loop(0, NCH, chunk_body, (0, 0, 0))

        g0, p0, p1 = ring

        @pl.when(wid == NW - 1)
        def _():
            pltpu.sync_copy(tail_hbm, tail_v)
            cst = pl.multiple_of(7808 * 128, 128)
            pltpu.sync_copy(tablet_hbm.at[:, pl.ds(cst, CHW)],
                            chunk_v.at[:, pl.ds(0, CHW)])
            r2 = extract_seg(chunk_v, cst, 0, segs_s[NCH], segs_s[NCH + 1],
                             (g0, p0, p1))
            r3 = extract_seg(tail_v, TAILC, 0, segs_s[NCH + 1],
                             segs_s[NCH + 2], r2)
            ga, pa, pb = r3

            def drain(k, c2):
                pltpu.make_async_copy(
                    stage_hbm.at[pl.ds(0, D)],
                    rowg_v.at[pl.ds(0, D)], sem_r).wait()
                return c2

            lax.fori_loop(0, pa + pb, drain, 0)

        @pl.when(wid != NW - 1)
        def _():
            def drain(k, c2):
                pltpu.make_async_copy(
                    stage_hbm.at[pl.ds(0, D)],
                    rowg_v.at[pl.ds(0, D)], sem_r).wait()
                return c2

            lax.fori_loop(0, p0 + p1, drain, 0)

        return it1, cnt

    lax.while_loop(lambda c: c[0] < NIT, round_body, (0, 0))


_extract = functools.partial(
    pl.kernel,
    out_type=jax.ShapeDtypeStruct((B2 * D,), jnp.float32),
    mesh=plsc.VectorSubcoreMesh(
        core_axis_name="c", subcore_axis_name="s",
        num_cores=NC, num_subcores=NS),
    scratch_types=[
        pltpu.VMEM((B2,), jnp.int32),
        pltpu.VMEM((CAP,), jnp.int32),
        pltpu.VMEM((CAP,), jnp.int32),
        pltpu.VMEM((CAP,), jnp.int32),
        pltpu.VMEM((CAP,), jnp.int32),
        pltpu.VMEM((D, 2 * CHW), jnp.float32),
        pltpu.VMEM((D, D), jnp.float32),
        pltpu.VMEM((2 * L * D,), jnp.float32),
        pltpu.SMEM((NBIN,), jnp.int32),
        pltpu.SMEM((NBIN + 1,), jnp.int32),
        pltpu.SMEM((NBIN,), jnp.int32),
        pltpu.SemaphoreType.DMA,
        pltpu.SemaphoreType.DMA,
        pltpu.SemaphoreType.DMA,
    ],
    compiler_params=pltpu.CompilerParams(needs_layout_passes=False),
)(_extract_body)


def _dot_body(stage_hbm, out_hbm, trows_v, crows_v, out_v):
    wid = lax.axis_index("s") * NC + lax.axis_index("c")
    base = wid * BPW

    pltpu.sync_copy(stage_hbm.at[pl.ds(base * D, BPW * D)], trows_v)
    pltpu.sync_copy(stage_hbm.at[pl.ds((B + base) * D, BPW * D)], crows_v)

    iota = lax.iota(jnp.int32, L)

    def blk_body(blk, carry):
        v = jnp.zeros((L,), jnp.float32)
        for j in range(L):
            r = blk * L + j
            s = jnp.zeros((L,), jnp.float32)
            for d in range(0, D, L):
                tv = trows_v[pl.ds(r * D + d, L)]
                cv = crows_v[pl.ds(r * D + d, L)]
                s = s + tv * cv
            v = jnp.where(iota == j, jnp.sum(s), v)
        out_v[pl.ds(blk * L, L)] = 1.0 / (1.0 + jnp.exp(-v))
        return carry

    lax.fori_loop(0, BPW // L, blk_body, 0)
    pltpu.sync_copy(out_v, out_hbm.at[pl.ds(base, BPW)])


_dot = functools.partial(
    pl.kernel,
    out_type=jax.ShapeDtypeStruct((B,), jnp.float32),
    mesh=plsc.VectorSubcoreMesh(
        core_axis_name="c", subcore_axis_name="s",
        num_cores=NC, num_subcores=NS),
    scratch_types=[
        pltpu.VMEM((BPW * D,), jnp.float32),
        pltpu.VMEM((BPW * D,), jnp.float32),
        pltpu.VMEM((BPW,), jnp.float32),
    ],
    compiler_params=pltpu.CompilerParams(needs_layout_passes=False),
)(_dot_body)


@jax.jit
def kernel(target_i, context_j, label, shared_embedding):
    table_t = shared_embedding.T
    tail = lax.slice(table_t, (0, TAILC), (D, V))
    stage = _extract(target_i, context_j, table_t, tail)
    out = _dot(stage)
    return (out, label.astype(jnp.float32))


# speedup vs baseline: 1.4529x; 1.0802x over previous
import functools

import jax
import jax.numpy as jnp
from jax import lax
from jax.experimental import pallas as pl
from jax.experimental.pallas import tpu as pltpu
from jax.experimental.pallas import tpu_sc as plsc

NC = 2
NS = 16
L = 16
NW = NC * NS

V = 1000000
B = 16384
B2 = 2 * B
D = 64
BPW = B // NW
NBLK = (V + 127) // 128
SLAB = 244
CHW = 512
NCH = 61
TAILC = 7812 * 128
CAP = 4096
NIT = B2 // L
NBIN = 64


def _extract_body(tgt_hbm, ctx_hbm, tablet_hbm, tail_hbm, stage_hbm,
                  qv, mv, mj, sv, sj, chunk_v, tail_v,
                  rowg_v, hist_s, segs_s, cur_s, sem_r, sem_c0, sem_c1):
    wid = lax.axis_index("s") * NC + lax.axis_index("c")
    sb = wid * SLAB
    sbc = sb * 128
    se = jnp.where(wid == NW - 1, NBLK, sb + SLAB)

    iota = lax.iota(jnp.int32, L)
    lane0 = iota == 0

    def extract_seg(cv, lo, cbase, s0, s1, ring):
        def group_body(e, ring):
            g0, p0, p1 = ring
            base = s0 + e * L
            ev = sv[pl.ds(base, L)]
            ej = sj[pl.ds(base, L)]
            nmc = jnp.minimum(s1 - base, L)
            vmask = iota < nmc
            h = g0 % 2
            rb = h * (L * D)
            pend = jnp.where(h == 0, p0, p1)

            def drain(k, c2):
                pltpu.make_async_copy(
                    stage_hbm.at[pl.ds(0, D)],
                    rowg_v.at[pl.ds(0, D)], sem_r).wait()
                return c2

            lax.fori_loop(0, pend, drain, 0)

            cols = ev - lo + cbase
            for f in range(D):
                f16 = jnp.full((L,), f, jnp.int32)
                g = plsc.load_gather(cv, [f16, cols], mask=vmask)
                plsc.store_scatter(rowg_v, [rb + iota * D + f], g, mask=vmask)

            def fire(k, c2):
                jk = ej[jnp.broadcast_to(k, (L,))][0]
                pltpu.async_copy(
                    rowg_v.at[pl.ds(rb + k * D, D)],
                    stage_hbm.at[pl.ds(jk * D, D)], sem_r)
                return c2

            lax.fori_loop(0, nmc, fire, 0)
            p0n = jnp.where(h == 0, nmc, p0)
            p1n = jnp.where(h == 1, nmc, p1)
            return g0 + 1, p0n, p1n

        ngrp = jnp.maximum(s1 - s0 + L - 1, 0) // L
        return lax.fori_loop(0, ngrp, group_body, ring)

    def round_body(carry):
        it0, _ = carry
        pltpu.async_copy(
            tablet_hbm.at[:, pl.ds(pl.multiple_of(sbc, 128), CHW)],
            chunk_v.at[:, pl.ds(0, CHW)], sem_c0)
        pltpu.async_copy(
            tablet_hbm.at[:, pl.ds(pl.multiple_of(sbc + CHW, 128), CHW)],
            chunk_v.at[:, pl.ds(CHW, CHW)], sem_c1)
        pltpu.sync_copy(tgt_hbm, qv.at[pl.ds(0, B)])
        pltpu.sync_copy(ctx_hbm, qv.at[pl.ds(B, B)])

        def scan_cond(c):
            it, cnt = c
            return (it < NIT) & (cnt <= CAP - L)

        def scan_step(c):
            it, cnt = c
            v16 = qv[pl.ds(it * L, L)]
            blk = lax.shift_right_logical(v16, 7)
            m = (blk >= sb) & (blk < se)
            plsc.store_compressed(mv.at[pl.ds(cnt, L)], v16, mask=m)
            plsc.store_compressed(
                mj.at[pl.ds(cnt, L)], it * L + iota, mask=m)
            return it + 1, cnt + plsc.all_reduce_population_count(m)[0]

        it1, cnt = lax.while_loop(scan_cond, scan_step, (it0, 0))

        sentinel = sbc + (63 << 9)
        mv[pl.ds(cnt, L)] = jnp.full((L,), sentinel, jnp.int32)
        mj[pl.ds(cnt, L)] = jnp.zeros((L,), jnp.int32)
        ngrp16 = (cnt + L - 1) // L

        def zero_bin(k, c2):
            hist_s[k] = 0
            return c2

        lax.fori_loop(0, NBIN, zero_bin, 0)

        def hist_group(e, c2):
            base = e * L
            ev = mv[pl.ds(base, L)]
            for l in range(L):
                b = lax.shift_right_logical(ev[l] - sbc, 9)
                hist_s[b] = hist_s[b] + 1
            return c2

        lax.fori_loop(0, ngrp16, hist_group, 0)

        def prefix(k, acc):
            segs_s[k] = acc
            cur_s[k] = acc
            return acc + hist_s[k]

        total = lax.fori_loop(0, NBIN, prefix, 0)
        segs_s[NBIN] = total

        def scat_group(e, c2):
            base = e * L
            ev = mv[pl.ds(base, L)]
            ej = mj[pl.ds(base, L)]
            for l in range(L):
                b = lax.shift_right_logical(ev[l] - sbc, 9)
                slot = cur_s[b]
                cur_s[b] = slot + 1
                plsc.store_scatter(
                    sv, [jnp.broadcast_to(slot, (L,))],
                    jnp.broadcast_to(ev[l], (L,)), mask=lane0)
                plsc.store_scatter(
                    sj, [jnp.broadcast_to(slot, (L,))],
                    jnp.broadcast_to(ej[l], (L,)), mask=lane0)
            return c2

        lax.fori_loop(0, ngrp16, scat_group, 0)

        def prefetch0(cc):
            cst = pl.multiple_of(sbc + cc * CHW, 128)
            pltpu.async_copy(tablet_hbm.at[:, pl.ds(cst, CHW)],
                             chunk_v.at[:, pl.ds(0, CHW)], sem_c0)

        def prefetch1(cc):
            cst = pl.multiple_of(sbc + cc * CHW, 128)
            pltpu.async_copy(tablet_hbm.at[:, pl.ds(cst, CHW)],
                             chunk_v.at[:, pl.ds(CHW, CHW)], sem_c1)

        def wait0():
            pltpu.make_async_copy(tablet_hbm.at[:, pl.ds(0, CHW)],
                                  chunk_v.at[:, pl.ds(0, CHW)], sem_c0).wait()

        def wait1():
            pltpu.make_async_copy(tablet_hbm.at[:, pl.ds(0, CHW)],
                                  chunk_v.at[:, pl.ds(CHW, CHW)], sem_c1).wait()

        def chunk_body(cc, ring):
            h = cc % 2
            cst = sbc + cc * CHW

            @pl.when(h == 0)
            def _():
                wait0()

            @pl.when(h == 1)
            def _():
                wait1()

            ring = extract_seg(chunk_v, cst, h * CHW,
                               segs_s[cc], segs_s[cc + 1], ring)

            @pl.when((cc + 2 < NCH) & (h == 0))
            def _():
                prefetch0(cc + 2)

            @pl.when((cc + 2 < NCH) & (h == 1))
            def _():
                prefetch1(cc + 2)

            return ring

        ring = lax.fori_loop(0, NCH, chunk_body, (0, 0, 0))

        g0, p0, p1 = ring

        @pl.when(wid == NW - 1)
        def _():
            pltpu.sync_copy(tail_hbm, tail_v)
            cst = pl.multiple_of(7808 * 128, 128)
            pltpu.sync_copy(tablet_hbm.at[:, pl.ds(cst, CHW)],
                            chunk_v.at[:, pl.ds(0, CHW)])
            r2 = extract_seg(chunk_v, cst, 0, segs_s[NCH], segs_s[NCH + 1],
                             (g0, p0, p1))
            r3 = extract_seg(tail_v, TAILC, 0, segs_s[NCH + 1],
                             segs_s[NCH + 2], r2)
            ga, pa, pb = r3

            def drain(k, c2):
                pltpu.make_async_copy(
                    stage_hbm.at[pl.ds(0, D)],
                    rowg_v.at[pl.ds(0, D)], sem_r).wait()
                return c2

            lax.fori_loop(0, pa + pb, drain, 0)

        @pl.when(wid != NW - 1)
        def _():
            def drain(k, c2):
                pltpu.make_async_copy(
                    stage_hbm.at[pl.ds(0, D)],
                    rowg_v.at[pl.ds(0, D)], sem_r).wait()
                return c2

            lax.fori_loop(0, p0 + p1, drain, 0)

        return it1, cnt

    lax.while_loop(lambda c: c[0] < NIT, round_body, (0, 0))


_extract = functools.partial(
    pl.kernel,
    out_type=jax.ShapeDtypeStruct((B2 * D,), jnp.float32),
    mesh=plsc.VectorSubcoreMesh(
        core_axis_name="c", subcore_axis_name="s",
        num_cores=NC, num_subcores=NS),
    scratch_types=[
        pltpu.VMEM((B2,), jnp.int32),
        pltpu.VMEM((CAP + L,), jnp.int32),
        pltpu.VMEM((CAP + L,), jnp.int32),
        pltpu.VMEM((CAP + L,), jnp.int32),
        pltpu.VMEM((CAP + L,), jnp.int32),
        pltpu.VMEM((D, 2 * CHW), jnp.float32),
        pltpu.VMEM((D, D), jnp.float32),
        pltpu.VMEM((2 * L * D,), jnp.float32),
        pltpu.SMEM((NBIN,), jnp.int32),
        pltpu.SMEM((NBIN + 1,), jnp.int32),
        pltpu.SMEM((NBIN,), jnp.int32),
        pltpu.SemaphoreType.DMA,
        pltpu.SemaphoreType.DMA,
        pltpu.SemaphoreType.DMA,
    ],
    compiler_params=pltpu.CompilerParams(needs_layout_passes=False),
)(_extract_body)


def _dot_body(stage_hbm, out_hbm, trows_v, crows_v, out_v):
    wid = lax.axis_index("s") * NC + lax.axis_index("c")
    base = wid * BPW

    pltpu.sync_copy(stage_hbm.at[pl.ds(base * D, BPW * D)], trows_v)
    pltpu.sync_copy(stage_hbm.at[pl.ds((B + base) * D, BPW * D)], crows_v)

    iota = lax.iota(jnp.int32, L)

    def blk_body(blk, carry):
        v = jnp.zeros((L,), jnp.float32)
        for j in range(L):
            r = blk * L + j
            s = jnp.zeros((L,), jnp.float32)
            for d in range(0, D, L):
                tv = trows_v[pl.ds(r * D + d, L)]
                cv = crows_v[pl.ds(r * D + d, L)]
                s = s + tv * cv
            v = jnp.where(iota == j, jnp.sum(s), v)
        out_v[pl.ds(blk * L, L)] = 1.0 / (1.0 + jnp.exp(-v))
        return carry

    lax.fori_loop(0, BPW // L, blk_body, 0)
    pltpu.sync_copy(out_v, out_hbm.at[pl.ds(base, BPW)])


_dot = functools.partial(
    pl.kernel,
    out_type=jax.ShapeDtypeStruct((B,), jnp.float32),
    mesh=plsc.VectorSubcoreMesh(
        core_axis_name="c", subcore_axis_name="s",
        num_cores=NC, num_subcores=NS),
    scratch_types=[
        pltpu.VMEM((BPW * D,), jnp.float32),
        pltpu.VMEM((BPW * D,), jnp.float32),
        pltpu.VMEM((BPW,), jnp.float32),
    ],
    compiler_params=pltpu.CompilerParams(needs_layout_passes=False),
)(_dot_body)


@jax.jit
def kernel(target_i, context_j, label, shared_embedding):
    table_t = shared_embedding.T
    tail = lax.slice(table_t, (0, TAILC), (D, V))
    stage = _extract(target_i, context_j, table_t, tail)
    out = _dot(stage)
    return (out, label.astype(jnp.float32))
